# baseline (device time: 815298 ns/iter reference)
import jax
import jax.numpy as jnp
from jax import lax
from jax.experimental import pallas as pl
from jax.experimental.pallas import tpu as pltpu

VROWS = 2048
NSLOTS = 2


def kernel(x):
    m_per, n = x.shape
    nch = m_per // VROWS

    def body(x_ref, out_ref, fbuf, bbuf, load_sems, store_sems, send_sems, recv_sems):
        my_x = lax.axis_index("x")
        my_y = lax.axis_index("y")
        my_z = lax.axis_index("z")
        nbr = (my_x, my_y, 1 - my_z)

        barrier = pltpu.get_barrier_semaphore()
        pl.semaphore_signal(
            barrier, inc=1, device_id=nbr, device_id_type=pl.DeviceIdType.MESH
        )
        pl.semaphore_wait(barrier, 1)

        base = my_z * m_per

        rdmas = []
        for c in range(nch):
            slot = c % NSLOTS

            rows = pl.ds(base + c * VROWS, VROWS)

            ld = pltpu.make_async_copy(
                x_ref.at[pl.ds(c * VROWS, VROWS), :],
                fbuf.at[slot],
                load_sems.at[slot],
            )
            ld.start()
            ld.wait()

            bbuf[slot] = fbuf[slot][...].astype(jnp.bfloat16)

            st = pltpu.make_async_copy(
                bbuf.at[slot], out_ref.at[rows, :], store_sems.at[slot]
            )
            st.start()
            st.wait()

            r = pltpu.make_async_remote_copy(
                src_ref=out_ref.at[rows, :],
                dst_ref=out_ref.at[rows, :],
                send_sem=send_sems.at[c],
                recv_sem=recv_sems.at[c],
                device_id=nbr,
                device_id_type=pl.DeviceIdType.MESH,
            )
            r.start()
            rdmas.append(r)

        for c in range(nch):
            rdmas[c].wait_send()
        for c in range(nch):
            rdmas[c].wait_recv()

    return pl.pallas_call(
        body,
        out_shape=jax.ShapeDtypeStruct((2 * m_per, n), jnp.bfloat16),
        in_specs=[pl.BlockSpec(memory_space=pl.ANY)],
        out_specs=pl.BlockSpec(memory_space=pl.ANY),
        scratch_shapes=[
            pltpu.VMEM((NSLOTS, VROWS, n), jnp.float32),
            pltpu.VMEM((NSLOTS, VROWS, n), jnp.bfloat16),
            pltpu.SemaphoreType.DMA((NSLOTS,)),
            pltpu.SemaphoreType.DMA((NSLOTS,)),
            pltpu.SemaphoreType.DMA((nch,)),
            pltpu.SemaphoreType.DMA((nch,)),
        ],
        compiler_params=pltpu.CompilerParams(collective_id=0),
    )(x)


# device time: 811703 ns/iter; 1.0044x vs baseline; 1.0044x over previous
import jax
import jax.numpy as jnp
from jax import lax
from jax.experimental import pallas as pl
from jax.experimental.pallas import tpu as pltpu

VROWS = 1024
NF = 2
NB = 6


def kernel(x):
    m_per, n = x.shape
    nch = m_per // VROWS

    def body(x_ref, out_ref, *scratch):
        fbufs = scratch[:NF]
        bbufs = scratch[NF : NF + NB]
        load_sems, store_sems, send_sems, recv_sems = scratch[NF + NB :]

        my_x = lax.axis_index("x")
        my_y = lax.axis_index("y")
        my_z = lax.axis_index("z")
        nbr = (my_x, my_y, 1 - my_z)

        barrier = pltpu.get_barrier_semaphore()
        pl.semaphore_signal(
            barrier, inc=1, device_id=nbr, device_id_type=pl.DeviceIdType.MESH
        )
        pl.semaphore_wait(barrier, 1)

        base = my_z * m_per

        rdmas = []
        stores = []
        for c in range(nch):
            f = c % NF
            b = c % NB

            ld = pltpu.make_async_copy(
                x_ref.at[pl.ds(c * VROWS, VROWS), :], fbufs[f], load_sems.at[f]
            )
            ld.start()
            ld.wait()

            if c >= NB:
                rdmas[c - NB].wait_send()
                stores[c - NB].wait()

            bbufs[b][...] = fbufs[f][...].astype(jnp.bfloat16)

            st = pltpu.make_async_copy(
                bbufs[b],
                out_ref.at[pl.ds(base + c * VROWS, VROWS), :],
                store_sems.at[b],
            )
            st.start()
            stores.append(st)

            r = pltpu.make_async_remote_copy(
                src_ref=bbufs[b],
                dst_ref=out_ref.at[pl.ds(base + c * VROWS, VROWS), :],
                send_sem=send_sems.at[c],
                recv_sem=recv_sems.at[c],
                device_id=nbr,
                device_id_type=pl.DeviceIdType.MESH,
            )
            r.start()
            rdmas.append(r)

        for c in range(nch - NB, nch):
            rdmas[c].wait_send()
            stores[c].wait()
        for c in range(nch):
            rdmas[c].wait_recv()

    return pl.pallas_call(
        body,
        out_shape=jax.ShapeDtypeStruct((2 * m_per, n), jnp.bfloat16),
        in_specs=[pl.BlockSpec(memory_space=pl.ANY)],
        out_specs=pl.BlockSpec(memory_space=pl.ANY),
        scratch_shapes=(
            [pltpu.VMEM((VROWS, n), jnp.float32) for _ in range(NF)]
            + [pltpu.VMEM((VROWS, n), jnp.bfloat16) for _ in range(NB)]
            + [
                pltpu.SemaphoreType.DMA((NF,)),
                pltpu.SemaphoreType.DMA((NB,)),
                pltpu.SemaphoreType.DMA((nch,)),
                pltpu.SemaphoreType.DMA((nch,)),
            ]
        ),
        compiler_params=pltpu.CompilerParams(collective_id=0),
    )(x)
